# baseline (speedup 1.0000x reference)
import functools

import jax
import jax.numpy as jnp
from jax import lax
from jax.experimental import pallas as pl
from jax.experimental.pallas import tpu as pltpu
from jax.experimental.pallas import tpu_sc as plsc

NDIM = 768
NROWS = 62
NC, NS = 2, 16
NW = NC * NS
CHUNK = 64


def _proj_body(t_ref, w_ref, p_ref):
    p_ref[...] = lax.dot_general(
        t_ref[...], w_ref[...], (((1,), (1,)), ((), ())),
        preferred_element_type=jnp.float32)


def _compute_proj(table, W):
    return pl.pallas_call(
        _proj_body,
        out_shape=jax.ShapeDtypeStruct((NROWS, NDIM), jnp.float32),
    )(table, W)


def _make_gather(b_total):
    assert b_total % (NW * CHUNK) == 0
    bpw = b_total // NW
    nchunk = bpw // CHUNK

    @functools.partial(
        pl.kernel,
        out_type=jax.ShapeDtypeStruct((b_total, NDIM), jnp.float32),
        mesh=plsc.VectorSubcoreMesh(
            core_axis_name="c", subcore_axis_name="s",
            num_cores=NC, num_subcores=NS),
        scratch_types=[
            pltpu.VMEM((bpw,), jnp.int32),
            pltpu.VMEM((CHUNK, NDIM), jnp.float32),
            pltpu.SemaphoreType.DMA,
        ],
    )
    def _gather(p_hbm, idx_hbm, out_hbm, idx_v, buf_v, sem):
        wid = lax.axis_index("s") * NC + lax.axis_index("c")
        base = wid * bpw
        pltpu.sync_copy(idx_hbm.at[pl.ds(base, bpw)], idx_v)

        def chunk_body(g, carry):
            off = pl.multiple_of(g * CHUNK, CHUNK)
            pltpu.async_copy(
                p_hbm.at[idx_v.at[pl.ds(off, CHUNK)]], buf_v, sem).wait()
            pltpu.sync_copy(buf_v, out_hbm.at[pl.ds(base + off, CHUNK)])
            return carry

        lax.fori_loop(0, nchunk, chunk_body, 0)

    return _gather


def kernel(x, table, W):
    b, l = x.shape
    p = _compute_proj(table, W)
    out = _make_gather(b * l)(p, x.reshape(-1))
    return out.reshape(b, l, NDIM)
